# baseline (speedup 1.0000x reference)
import functools

import jax
import jax.numpy as jnp
from jax import lax
from jax.experimental import pallas as pl
from jax.experimental.pallas import tpu as pltpu
from jax.experimental.pallas import tpu_sc as plsc

N = 10000
E = 320000
F_IN = 128
H = 128
C = 40
G = 8
SEG = N // G

NC = 2
NS = 16
NW = NC * NS
EPW = E // NW
K = 80
NCH = EPW // K
NV = K // 16
UN = N // K
UPT = (UN + NS - 1) // NS

_MESH = plsc.VectorSubcoreMesh(
    core_axis_name="c", subcore_axis_name="s", num_cores=NC, num_subcores=NS)


def _unpack_chunk(pk_v, j, src_k, dst_k):
    for l in range(NV):
        v = pk_v[j, pl.ds(l * 16, 16)]
        src_k[pl.ds(l * 16, 16)] = lax.shift_right_logical(v, 16)
        dst_k[pl.ds(l * 16, 16)] = lax.bitwise_and(v, 0xFFFF)


@functools.partial(
    pl.kernel,
    out_type=jax.ShapeDtypeStruct((NC * N,), jnp.float32),
    mesh=_MESH,
    scratch_types=[
        pltpu.VMEM((NCH, K), jnp.int32),
        pltpu.VMEM((NCH, K), jnp.int32),
        pltpu.VMEM((K,), jnp.float32),
        pltpu.VMEM((K,), jnp.float32),
        pltpu.VMEM_SHARED((N,), jnp.float32),
        pltpu.SemaphoreType.DMA,
    ],
)
def _deg_kernel(edge_hbm, ones_hbm, zeros_hbm, out_hbm, pk_v, dst_v,
                ones_v, stage_v, acc, sem):
    c = lax.axis_index("c")
    s = lax.axis_index("s")
    wid = s * NC + c
    pltpu.sync_copy(edge_hbm.at[wid], pk_v)
    pltpu.sync_copy(ones_hbm, ones_v)
    pltpu.sync_copy(zeros_hbm, stage_v)

    def ubody(j, carry):
        for l in range(NV):
            v = pk_v[j, pl.ds(l * 16, 16)]
            dst_v[j, pl.ds(l * 16, 16)] = lax.bitwise_and(v, 0xFFFF)
        return carry

    lax.fori_loop(0, NCH, ubody, 0)

    for u_i in range(UPT):
        u = s + u_i * NS

        @pl.when(u < UN)
        def _():
            start = pl.multiple_of(u * K, K)
            pltpu.sync_copy(stage_v, acc.at[pl.ds(start, K)])

    plsc.subcore_barrier()

    def body(j, carry):
        pltpu.async_copy(ones_v, acc.at[dst_v.at[j]], sem, add=True)
        return carry

    lax.fori_loop(0, NCH, body, 0)

    def dbody(j, carry):
        pltpu.make_async_copy(ones_v, acc.at[dst_v.at[j]], sem).wait()
        return carry

    lax.fori_loop(0, NCH, dbody, 0)
    plsc.subcore_barrier()

    for u_i in range(UPT):
        u = s + u_i * NS

        @pl.when(u < UN)
        def _():
            start = pl.multiple_of(u * K, K)
            ostart = pl.multiple_of(c * N + u * K, K)
            pltpu.sync_copy(acc.at[pl.ds(start, K)], stage_v)
            pltpu.sync_copy(stage_v, out_hbm.at[pl.ds(ostart, K)])


@functools.partial(
    pl.kernel,
    out_type=jax.ShapeDtypeStruct((NC, N, H), jnp.float32),
    mesh=_MESH,
    scratch_types=[
        pltpu.VMEM((NCH, K), jnp.int32),
        [pltpu.VMEM((K,), jnp.int32)] * 3,
        [pltpu.VMEM((K,), jnp.int32)] * 3,
        [pltpu.VMEM((K, H), jnp.float32)] * 3,
        pltpu.VMEM_SHARED((N, H), jnp.float32),
        [pltpu.SemaphoreType.DMA] * 3,
        [pltpu.SemaphoreType.DMA] * 3,
    ],
)
def _scatter_kernel(xw_hbm, edge_hbm, out_hbm,
                    pk_v, srcs, dsts, rows, acc, gsems, ssems):
    c = lax.axis_index("c")
    s = lax.axis_index("s")
    wid = s * NC + c
    pltpu.sync_copy(edge_hbm.at[wid], pk_v)

    def zbody(i, carry):
        for l in range(H // 16):
            rows[0][i, pl.ds(l * 16, 16)] = jnp.zeros((16,), jnp.float32)
        return carry

    lax.fori_loop(0, K, zbody, 0)

    for u_i in range(UPT):
        u = s + u_i * NS

        @pl.when(u < UN)
        def _():
            start = pl.multiple_of(u * K, K)
            pltpu.sync_copy(rows[0], acc.at[pl.ds(start, K)])

    plsc.subcore_barrier()

    def gather(x, j):
        _unpack_chunk(pk_v, j, srcs[x], dsts[x])
        pltpu.async_copy(xw_hbm.at[srcs[x]], rows[x], gsems[x])

    def wait_gather(x):
        pltpu.make_async_copy(xw_hbm.at[srcs[x]], rows[x], gsems[x]).wait()

    def scatter(x):
        pltpu.async_copy(rows[x], acc.at[dsts[x]], ssems[x], add=True)

    def wait_scatter(x):
        pltpu.make_async_copy(rows[x], acc.at[dsts[x]], ssems[x]).wait()

    for x in range(3):
        gather(x, x)

    def body3(t, carry):
        j = 3 * t
        for x in range(3):
            wait_gather(x)
            scatter(x)
        for x in range(3):
            wait_scatter(x)
            gather(x, j + 3 + x)
        return carry

    lax.fori_loop(0, (NCH - 5) // 3, body3, 0)
    for x in range(3):
        wait_gather(x)
        scatter(x)
    for x in range(2):
        wait_scatter(x)
        gather(x, NCH - 2 + x)
    for x in range(2):
        wait_gather(x)
        scatter(x)
    for x in range(3):
        wait_scatter(x)
    plsc.subcore_barrier()

    for u_i in range(UPT):
        u = s + u_i * NS

        @pl.when(u < UN)
        def _():
            start = pl.multiple_of(u * K, K)
            pltpu.sync_copy(acc.at[pl.ds(start, K)], rows[0])
            pltpu.sync_copy(rows[0], out_hbm.at[c, pl.ds(start, K)])


def _seg_body(em_ref, x_ref, w0_ref, b0_ref, w1_ref, xw_ref):
    nw0 = jnp.maximum(w0_ref[...] * em_ref[0] + b0_ref[...], 0.0)
    h0 = jnp.dot(x_ref[0], nw0, preferred_element_type=jnp.float32)
    xw_ref[0] = jnp.dot(h0, w1_ref[...], preferred_element_type=jnp.float32)


def _scale_body(degp_ref, xw_ref, xwp_ref, dinv_ref):
    deg = degp_ref[0] + degp_ref[1] + 1.0
    dinv = lax.rsqrt(deg)
    xwp_ref[...] = xw_ref[...] * dinv
    dinv_ref[...] = dinv


def _epi_body(acc_ref, xwp_ref, dinv_ref, b_ref, w2_ref, out_ref):
    dinv = dinv_ref[...]
    tot = acc_ref[0] + acc_ref[1] + xwp_ref[...]
    h = jnp.maximum(dinv * tot + b_ref[...], 0.0)
    xw2 = jnp.dot(h, w2_ref[...], preferred_element_type=jnp.float32)
    out_ref[...] = xw2 * dinv


def _fin_body(acc_ref, xwp_ref, dinv_ref, b_ref, wc_ref, bc_ref, out_ref):
    dinv = dinv_ref[...]
    tot = acc_ref[0] + acc_ref[1] + xwp_ref[...]
    h = jnp.maximum(dinv * tot + b_ref[...], 0.0)
    logits = jnp.dot(h, wc_ref[...], preferred_element_type=jnp.float32)
    logits = logits + bc_ref[...]
    m = jnp.max(logits, axis=1, keepdims=True)
    z = logits - m
    lse = jnp.log(jnp.sum(jnp.exp(z), axis=1, keepdims=True))
    out_ref[...] = (z - lse)[:, :C]


def kernel(x, edge_index, E_meta, ptr, w0, b0, conv_W, conv_b, lt1_W, lt1_b):
    del ptr

    packed = (jnp.left_shift(edge_index[0], 16) | edge_index[1])
    edges3 = packed.reshape(NW, NCH, K)
    ones_k = jnp.ones((K,), jnp.float32)
    zeros_k = jnp.zeros((K,), jnp.float32)

    deg_p = _deg_kernel(edges3, ones_k, zeros_k)
    degp3 = deg_p.reshape(NC, N, 1)

    x3 = x.reshape(G, SEG, F_IN)
    em3 = E_meta.reshape(G, 1, H)
    xw1 = pl.pallas_call(
        _seg_body,
        grid=(G,),
        in_specs=[
            pl.BlockSpec((1, 1, H), lambda i: (i, 0, 0)),
            pl.BlockSpec((1, SEG, F_IN), lambda i: (i, 0, 0)),
            pl.BlockSpec((F_IN, 1), lambda i: (0, 0)),
            pl.BlockSpec((F_IN, H), lambda i: (0, 0)),
            pl.BlockSpec((H, H), lambda i: (0, 0)),
        ],
        out_specs=pl.BlockSpec((1, SEG, H), lambda i: (i, 0, 0)),
        out_shape=jax.ShapeDtypeStruct((G, SEG, H), jnp.float32),
    )(em3, x3, w0, b0, conv_W[0])
    xw1 = xw1.reshape(N, H)

    RB = 2000
    NRB = N // RB
    xwp1, dinv = pl.pallas_call(
        _scale_body,
        grid=(NRB,),
        in_specs=[
            pl.BlockSpec((NC, RB, 1), lambda i: (0, i, 0)),
            pl.BlockSpec((RB, H), lambda i: (i, 0)),
        ],
        out_specs=[
            pl.BlockSpec((RB, H), lambda i: (i, 0)),
            pl.BlockSpec((RB, 1), lambda i: (i, 0)),
        ],
        out_shape=[
            jax.ShapeDtypeStruct((N, H), jnp.float32),
            jax.ShapeDtypeStruct((N, 1), jnp.float32),
        ],
    )(degp3, xw1)

    acc1 = _scatter_kernel(xwp1, edges3)

    b0row = conv_b[0].reshape(1, H)
    epi_specs = [
        pl.BlockSpec((NC, RB, H), lambda i: (0, i, 0)),
        pl.BlockSpec((RB, H), lambda i: (i, 0)),
        pl.BlockSpec((RB, 1), lambda i: (i, 0)),
        pl.BlockSpec((1, H), lambda i: (0, 0)),
        pl.BlockSpec((H, H), lambda i: (0, 0)),
    ]
    xwp2 = pl.pallas_call(
        _epi_body,
        grid=(NRB,),
        in_specs=epi_specs,
        out_specs=pl.BlockSpec((RB, H), lambda i: (i, 0)),
        out_shape=jax.ShapeDtypeStruct((N, H), jnp.float32),
    )(acc1, xwp1, dinv, b0row, conv_W[1])

    acc2 = _scatter_kernel(xwp2, edges3)

    b1row = conv_b[1].reshape(1, H)
    wc = jnp.pad(lt1_W, ((0, 0), (0, H - C)))
    bc = jnp.pad(lt1_b, (0, H - C), constant_values=-1e30).reshape(1, H)
    out = pl.pallas_call(
        _fin_body,
        grid=(NRB,),
        in_specs=epi_specs + [pl.BlockSpec((1, H), lambda i: (0, 0))],
        out_specs=pl.BlockSpec((RB, C), lambda i: (i, 0)),
        out_shape=jax.ShapeDtypeStruct((N, C), jnp.float32),
    )(acc2, xwp2, dinv, b1row, wc, bc)
    return out


# speedup vs baseline: 1.0368x; 1.0368x over previous
import functools

import jax
import jax.numpy as jnp
from jax import lax
from jax.experimental import pallas as pl
from jax.experimental.pallas import tpu as pltpu
from jax.experimental.pallas import tpu_sc as plsc

N = 10000
E = 320000
F_IN = 128
H = 128
C = 40
G = 8
SEG = N // G

NC = 2
NS = 16
NW = NC * NS
EPW = E // NW
K = 80
NCH = EPW // K
NV = K // 16
UN = N // K
UPT = (UN + NS - 1) // NS

_MESH = plsc.VectorSubcoreMesh(
    core_axis_name="c", subcore_axis_name="s", num_cores=NC, num_subcores=NS)


def _unpack_chunk(pk_v, j, src_k, dst_k):
    for l in range(NV):
        v = pk_v[j, pl.ds(l * 16, 16)]
        src_k[pl.ds(l * 16, 16)] = lax.shift_right_logical(v, 16)
        dst_k[pl.ds(l * 16, 16)] = lax.bitwise_and(v, 0xFFFF)


@functools.partial(
    pl.kernel,
    out_type=jax.ShapeDtypeStruct((NC * N,), jnp.float32),
    mesh=_MESH,
    scratch_types=[
        pltpu.VMEM((NCH, K), jnp.int32),
        pltpu.VMEM((NCH, K), jnp.int32),
        pltpu.VMEM((K,), jnp.float32),
        pltpu.VMEM((K,), jnp.float32),
        pltpu.VMEM_SHARED((N,), jnp.float32),
        pltpu.SemaphoreType.DMA,
    ],
)
def _deg_kernel(edge_hbm, ones_hbm, zeros_hbm, out_hbm, pk_v, dst_v,
                ones_v, stage_v, acc, sem):
    c = lax.axis_index("c")
    s = lax.axis_index("s")
    wid = s * NC + c
    pltpu.sync_copy(edge_hbm.at[wid], pk_v)
    pltpu.sync_copy(ones_hbm, ones_v)
    pltpu.sync_copy(zeros_hbm, stage_v)

    def ubody(j, carry):
        for l in range(NV):
            v = pk_v[j, pl.ds(l * 16, 16)]
            dst_v[j, pl.ds(l * 16, 16)] = lax.bitwise_and(v, 0xFFFF)
        return carry

    lax.fori_loop(0, NCH, ubody, 0)

    for u_i in range(UPT):
        u = s + u_i * NS

        @pl.when(u < UN)
        def _():
            start = pl.multiple_of(u * K, K)
            pltpu.sync_copy(stage_v, acc.at[pl.ds(start, K)])

    plsc.subcore_barrier()

    def body(j, carry):
        pltpu.async_copy(ones_v, acc.at[dst_v.at[j]], sem, add=True)
        return carry

    lax.fori_loop(0, NCH, body, 0)

    def dbody(j, carry):
        pltpu.make_async_copy(ones_v, acc.at[dst_v.at[j]], sem).wait()
        return carry

    lax.fori_loop(0, NCH, dbody, 0)
    plsc.subcore_barrier()

    for u_i in range(UPT):
        u = s + u_i * NS

        @pl.when(u < UN)
        def _():
            start = pl.multiple_of(u * K, K)
            ostart = pl.multiple_of(c * N + u * K, K)
            pltpu.sync_copy(acc.at[pl.ds(start, K)], stage_v)
            pltpu.sync_copy(stage_v, out_hbm.at[pl.ds(ostart, K)])


@functools.partial(
    pl.kernel,
    out_type=jax.ShapeDtypeStruct((NC, N, H), jnp.float32),
    mesh=_MESH,
    scratch_types=[
        pltpu.VMEM((NCH, K), jnp.int32),
        [pltpu.VMEM((K,), jnp.int32)] * 3,
        [pltpu.VMEM((K,), jnp.int32)] * 3,
        [pltpu.VMEM((K, H), jnp.float32)] * 3,
        pltpu.VMEM_SHARED((N, H), jnp.float32),
        [pltpu.SemaphoreType.DMA] * 3,
        [pltpu.SemaphoreType.DMA] * 3,
    ],
)
def _scatter_kernel(xw_hbm, edge_hbm, out_hbm,
                    pk_v, srcs, dsts, rows, acc, gsems, ssems):
    c = lax.axis_index("c")
    s = lax.axis_index("s")
    wid = s * NC + c
    pltpu.sync_copy(edge_hbm.at[wid], pk_v)

    def zbody(i, carry):
        for l in range(H // 16):
            rows[0][i, pl.ds(l * 16, 16)] = jnp.zeros((16,), jnp.float32)
        return carry

    lax.fori_loop(0, K, zbody, 0)

    for u_i in range(UPT):
        u = s + u_i * NS

        @pl.when(u < UN)
        def _():
            start = pl.multiple_of(u * K, K)
            pltpu.async_copy(rows[0], acc.at[pl.ds(start, K)], ssems[0])

    for u_i in range(UPT):
        u = s + u_i * NS

        @pl.when(u < UN)
        def _():
            start = pl.multiple_of(u * K, K)
            pltpu.make_async_copy(rows[0], acc.at[pl.ds(start, K)],
                                  ssems[0]).wait()

    plsc.subcore_barrier()

    def gather(x, j):
        _unpack_chunk(pk_v, j, srcs[x], dsts[x])
        pltpu.async_copy(xw_hbm.at[srcs[x]], rows[x], gsems[x])

    def wait_gather(x):
        pltpu.make_async_copy(xw_hbm.at[srcs[x]], rows[x], gsems[x]).wait()

    def scatter(x):
        pltpu.async_copy(rows[x], acc.at[dsts[x]], ssems[x], add=True)

    def wait_scatter(x):
        pltpu.make_async_copy(rows[x], acc.at[dsts[x]], ssems[x]).wait()

    for x in range(3):
        gather(x, x)

    def body3(t, carry):
        j = 3 * t
        for x in range(3):
            wait_gather(x)
            scatter(x)
        for x in range(3):
            wait_scatter(x)
            gather(x, j + 3 + x)
        return carry

    lax.fori_loop(0, (NCH - 5) // 3, body3, 0)
    for x in range(3):
        wait_gather(x)
        scatter(x)
    for x in range(2):
        wait_scatter(x)
        gather(x, NCH - 2 + x)
    for x in range(2):
        wait_gather(x)
        scatter(x)
    for x in range(3):
        wait_scatter(x)
    plsc.subcore_barrier()

    def dr_start(u_i):
        u = s + u_i * NS

        @pl.when(u < UN)
        def _():
            start = pl.multiple_of(u * K, K)
            x = u_i % 3
            pltpu.async_copy(acc.at[pl.ds(start, K)], rows[x], gsems[x])

    def dr_finish(u_i):
        u = s + u_i * NS

        @pl.when(u < UN)
        def _():
            start = pl.multiple_of(u * K, K)
            x = u_i % 3
            pltpu.make_async_copy(acc.at[pl.ds(start, K)], rows[x],
                                  gsems[x]).wait()
            pltpu.async_copy(rows[x], out_hbm.at[c, pl.ds(start, K)],
                             ssems[x])

    def dr_wait_write(u_i):
        u = s + u_i * NS

        @pl.when(u < UN)
        def _():
            start = pl.multiple_of(u * K, K)
            x = u_i % 3
            pltpu.make_async_copy(rows[x], out_hbm.at[c, pl.ds(start, K)],
                                  ssems[x]).wait()

    for u_i in range(min(3, UPT)):
        dr_start(u_i)
    for u_i in range(UPT):
        dr_finish(u_i)
        if u_i + 3 < UPT:
            dr_wait_write(u_i)
            dr_start(u_i + 3)
    for u_i in range(max(0, UPT - 3), UPT):
        dr_wait_write(u_i)


def _seg_body(em_ref, x_ref, degp_ref, w0_ref, b0_ref, w1_ref,
              xwp_ref, dinv_ref):
    nw0 = jnp.maximum(w0_ref[...] * em_ref[0] + b0_ref[...], 0.0)
    h0 = jnp.dot(x_ref[0], nw0, preferred_element_type=jnp.float32)
    xw = jnp.dot(h0, w1_ref[...], preferred_element_type=jnp.float32)
    deg = degp_ref[0, 0] + degp_ref[1, 0] + 1.0
    dinv = lax.rsqrt(deg)
    xwp_ref[0] = xw * dinv
    dinv_ref[0] = dinv


def _epi_body(acc_ref, xwp_ref, dinv_ref, b_ref, w2_ref, out_ref):
    dinv = dinv_ref[...]
    tot = acc_ref[0] + acc_ref[1] + xwp_ref[...]
    h = jnp.maximum(dinv * tot + b_ref[...], 0.0)
    xw2 = jnp.dot(h, w2_ref[...], preferred_element_type=jnp.float32)
    out_ref[...] = xw2 * dinv


def _fin_body(acc_ref, xwp_ref, dinv_ref, b_ref, wc_ref, bc_ref, out_ref):
    dinv = dinv_ref[...]
    tot = acc_ref[0] + acc_ref[1] + xwp_ref[...]
    h = jnp.maximum(dinv * tot + b_ref[...], 0.0)
    logits = jnp.dot(h, wc_ref[...], preferred_element_type=jnp.float32)
    logits = logits + bc_ref[...]
    m = jnp.max(logits, axis=1, keepdims=True)
    z = logits - m
    lse = jnp.log(jnp.sum(jnp.exp(z), axis=1, keepdims=True))
    out_ref[...] = (z - lse)[:, :C]


def kernel(x, edge_index, E_meta, ptr, w0, b0, conv_W, conv_b, lt1_W, lt1_b):
    del ptr

    packed = (jnp.left_shift(edge_index[0], 16) | edge_index[1])
    edges3 = packed.reshape(NW, NCH, K)
    ones_k = jnp.ones((K,), jnp.float32)
    zeros_k = jnp.zeros((K,), jnp.float32)

    deg_p = _deg_kernel(edges3, ones_k, zeros_k)
    degp4 = deg_p.reshape(NC, G, SEG, 1)

    x3 = x.reshape(G, SEG, F_IN)
    em3 = E_meta.reshape(G, 1, H)
    xwp1, dinv = pl.pallas_call(
        _seg_body,
        grid=(G,),
        in_specs=[
            pl.BlockSpec((1, 1, H), lambda i: (i, 0, 0)),
            pl.BlockSpec((1, SEG, F_IN), lambda i: (i, 0, 0)),
            pl.BlockSpec((NC, 1, SEG, 1), lambda i: (0, i, 0, 0)),
            pl.BlockSpec((F_IN, 1), lambda i: (0, 0)),
            pl.BlockSpec((F_IN, H), lambda i: (0, 0)),
            pl.BlockSpec((H, H), lambda i: (0, 0)),
        ],
        out_specs=[
            pl.BlockSpec((1, SEG, H), lambda i: (i, 0, 0)),
            pl.BlockSpec((1, SEG, 1), lambda i: (i, 0, 0)),
        ],
        out_shape=[
            jax.ShapeDtypeStruct((G, SEG, H), jnp.float32),
            jax.ShapeDtypeStruct((G, SEG, 1), jnp.float32),
        ],
    )(em3, x3, degp4, w0, b0, conv_W[0])
    xwp1 = xwp1.reshape(N, H)
    dinv = dinv.reshape(N, 1)

    RB = 2000
    NRB = N // RB

    acc1 = _scatter_kernel(xwp1, edges3)

    b0row = conv_b[0].reshape(1, H)
    epi_specs = [
        pl.BlockSpec((NC, RB, H), lambda i: (0, i, 0)),
        pl.BlockSpec((RB, H), lambda i: (i, 0)),
        pl.BlockSpec((RB, 1), lambda i: (i, 0)),
        pl.BlockSpec((1, H), lambda i: (0, 0)),
        pl.BlockSpec((H, H), lambda i: (0, 0)),
    ]
    xwp2 = pl.pallas_call(
        _epi_body,
        grid=(NRB,),
        in_specs=epi_specs,
        out_specs=pl.BlockSpec((RB, H), lambda i: (i, 0)),
        out_shape=jax.ShapeDtypeStruct((N, H), jnp.float32),
    )(acc1, xwp1, dinv, b0row, conv_W[1])

    acc2 = _scatter_kernel(xwp2, edges3)

    b1row = conv_b[1].reshape(1, H)
    wc = jnp.pad(lt1_W, ((0, 0), (0, H - C)))
    bc = jnp.pad(lt1_b, (0, H - C), constant_values=-1e30).reshape(1, H)
    out = pl.pallas_call(
        _fin_body,
        grid=(NRB,),
        in_specs=epi_specs + [pl.BlockSpec((1, H), lambda i: (0, 0))],
        out_specs=pl.BlockSpec((RB, C), lambda i: (i, 0)),
        out_shape=jax.ShapeDtypeStruct((N, C), jnp.float32),
    )(acc2, xwp2, dinv, b1row, wc, bc)
    return out


# speedup vs baseline: 1.0373x; 1.0005x over previous
import functools

import jax
import jax.numpy as jnp
from jax import lax
from jax.experimental import pallas as pl
from jax.experimental.pallas import tpu as pltpu
from jax.experimental.pallas import tpu_sc as plsc

N = 10000
E = 320000
F_IN = 128
H = 128
C = 40
G = 8
SEG = N // G

NC = 2
NS = 16
NW = NC * NS
EPW = E // NW
K = 80
NCH = EPW // K
NV = K // 16
UN = N // K
UPT = (UN + NS - 1) // NS

@functools.lru_cache(maxsize=None)
def _mesh():
    return plsc.VectorSubcoreMesh(
        core_axis_name="c", subcore_axis_name="s",
        num_cores=NC, num_subcores=NS)


def _unpack_chunk(pk_v, j, src_k, dst_k):
    for l in range(NV):
        v = pk_v[j, pl.ds(l * 16, 16)]
        src_k[pl.ds(l * 16, 16)] = lax.shift_right_logical(v, 16)
        dst_k[pl.ds(l * 16, 16)] = lax.bitwise_and(v, 0xFFFF)


@functools.lru_cache(maxsize=None)
def _make_deg_kernel():
    return functools.partial(
        pl.kernel,
        out_type=jax.ShapeDtypeStruct((NC * N,), jnp.float32),
        mesh=_mesh(),
        scratch_types=[
            pltpu.VMEM((NCH, K), jnp.int32),
            pltpu.VMEM((NCH, K), jnp.int32),
            pltpu.VMEM((K,), jnp.float32),
            pltpu.VMEM((K,), jnp.float32),
            pltpu.VMEM_SHARED((N,), jnp.float32),
            pltpu.SemaphoreType.DMA,
        ],
    )(_deg_body)


def _deg_body(edge_hbm, ones_hbm, zeros_hbm, out_hbm, pk_v, dst_v,
              ones_v, stage_v, acc, sem):
    c = lax.axis_index("c")
    s = lax.axis_index("s")
    wid = s * NC + c
    pltpu.sync_copy(edge_hbm.at[wid], pk_v)
    pltpu.sync_copy(ones_hbm, ones_v)
    pltpu.sync_copy(zeros_hbm, stage_v)

    def ubody(j, carry):
        for l in range(NV):
            v = pk_v[j, pl.ds(l * 16, 16)]
            dst_v[j, pl.ds(l * 16, 16)] = lax.bitwise_and(v, 0xFFFF)
        return carry

    lax.fori_loop(0, NCH, ubody, 0)

    for u_i in range(UPT):
        u = s + u_i * NS

        @pl.when(u < UN)
        def _():
            start = pl.multiple_of(u * K, K)
            pltpu.sync_copy(stage_v, acc.at[pl.ds(start, K)])

    plsc.subcore_barrier()

    def body(j, carry):
        pltpu.async_copy(ones_v, acc.at[dst_v.at[j]], sem, add=True)
        return carry

    lax.fori_loop(0, NCH, body, 0)

    def dbody(j, carry):
        pltpu.make_async_copy(ones_v, acc.at[dst_v.at[j]], sem).wait()
        return carry

    lax.fori_loop(0, NCH, dbody, 0)
    plsc.subcore_barrier()

    for u_i in range(UPT):
        u = s + u_i * NS

        @pl.when(u < UN)
        def _():
            start = pl.multiple_of(u * K, K)
            ostart = pl.multiple_of(c * N + u * K, K)
            pltpu.sync_copy(acc.at[pl.ds(start, K)], stage_v)
            pltpu.sync_copy(stage_v, out_hbm.at[pl.ds(ostart, K)])


@functools.lru_cache(maxsize=None)
def _make_scatter_kernel():
    return functools.partial(
        pl.kernel,
        out_type=jax.ShapeDtypeStruct((NC, N, H), jnp.float32),
        mesh=_mesh(),
        scratch_types=[
            pltpu.VMEM((NCH, K), jnp.int32),
            [pltpu.VMEM((K,), jnp.int32)] * 3,
            [pltpu.VMEM((K,), jnp.int32)] * 3,
            [pltpu.VMEM((K, H), jnp.float32)] * 3,
            pltpu.VMEM_SHARED((N, H), jnp.float32),
            [pltpu.SemaphoreType.DMA] * 3,
            [pltpu.SemaphoreType.DMA] * 3,
        ],
    )(_scatter_body)


def _scatter_body(xw_hbm, edge_hbm, out_hbm,
                  pk_v, srcs, dsts, rows, acc, gsems, ssems):
    c = lax.axis_index("c")
    s = lax.axis_index("s")
    wid = s * NC + c
    pltpu.sync_copy(edge_hbm.at[wid], pk_v)

    def zbody(i, carry):
        for l in range(H // 16):
            rows[0][i, pl.ds(l * 16, 16)] = jnp.zeros((16,), jnp.float32)
        return carry

    lax.fori_loop(0, K, zbody, 0)

    for u_i in range(UPT):
        u = s + u_i * NS

        @pl.when(u < UN)
        def _():
            start = pl.multiple_of(u * K, K)
            pltpu.async_copy(rows[0], acc.at[pl.ds(start, K)], ssems[0])

    for u_i in range(UPT):
        u = s + u_i * NS

        @pl.when(u < UN)
        def _():
            start = pl.multiple_of(u * K, K)
            pltpu.make_async_copy(rows[0], acc.at[pl.ds(start, K)],
                                  ssems[0]).wait()

    plsc.subcore_barrier()

    def gather(x, j):
        _unpack_chunk(pk_v, j, srcs[x], dsts[x])
        pltpu.async_copy(xw_hbm.at[srcs[x]], rows[x], gsems[x])

    def wait_gather(x):
        pltpu.make_async_copy(xw_hbm.at[srcs[x]], rows[x], gsems[x]).wait()

    def scatter(x):
        pltpu.async_copy(rows[x], acc.at[dsts[x]], ssems[x], add=True)

    def wait_scatter(x):
        pltpu.make_async_copy(rows[x], acc.at[dsts[x]], ssems[x]).wait()

    for x in range(3):
        gather(x, x)

    def body3(t, carry):
        j = 3 * t
        for x in range(3):
            wait_gather(x)
            scatter(x)
        for x in range(3):
            wait_scatter(x)
            gather(x, j + 3 + x)
        return carry

    lax.fori_loop(0, (NCH - 5) // 3, body3, 0)
    for x in range(3):
        wait_gather(x)
        scatter(x)
    for x in range(2):
        wait_scatter(x)
        gather(x, NCH - 2 + x)
    for x in range(2):
        wait_gather(x)
        scatter(x)
    for x in range(3):
        wait_scatter(x)
    plsc.subcore_barrier()

    def dr_start(u_i):
        u = s + u_i * NS

        @pl.when(u < UN)
        def _():
            start = pl.multiple_of(u * K, K)
            x = u_i % 3
            pltpu.async_copy(acc.at[pl.ds(start, K)], rows[x], gsems[x])

    def dr_finish(u_i):
        u = s + u_i * NS

        @pl.when(u < UN)
        def _():
            start = pl.multiple_of(u * K, K)
            x = u_i % 3
            pltpu.make_async_copy(acc.at[pl.ds(start, K)], rows[x],
                                  gsems[x]).wait()
            pltpu.async_copy(rows[x], out_hbm.at[c, pl.ds(start, K)],
                             ssems[x])

    def dr_wait_write(u_i):
        u = s + u_i * NS

        @pl.when(u < UN)
        def _():
            start = pl.multiple_of(u * K, K)
            x = u_i % 3
            pltpu.make_async_copy(rows[x], out_hbm.at[c, pl.ds(start, K)],
                                  ssems[x]).wait()

    for u_i in range(min(3, UPT)):
        dr_start(u_i)
    for u_i in range(UPT):
        dr_finish(u_i)
        if u_i + 3 < UPT:
            dr_wait_write(u_i)
            dr_start(u_i + 3)
    for u_i in range(max(0, UPT - 3), UPT):
        dr_wait_write(u_i)


def _seg_body(em_ref, x_ref, degp_ref, w0_ref, b0_ref, w1_ref,
              xwp_ref, dinv_ref):
    nw0 = jnp.maximum(w0_ref[...] * em_ref[0] + b0_ref[...], 0.0)
    h0 = jnp.dot(x_ref[0], nw0, preferred_element_type=jnp.float32)
    xw = jnp.dot(h0, w1_ref[...], preferred_element_type=jnp.float32)
    deg = degp_ref[0, 0] + degp_ref[1, 0] + 1.0
    dinv = lax.rsqrt(deg)
    xwp_ref[0] = xw * dinv
    dinv_ref[0] = dinv


def _epi_body(acc_ref, xwp_ref, dinv_ref, b_ref, w2_ref, out_ref):
    dinv = dinv_ref[...]
    tot = acc_ref[0] + acc_ref[1] + xwp_ref[...]
    h = jnp.maximum(dinv * tot + b_ref[...], 0.0)
    xw2 = jnp.dot(h, w2_ref[...], preferred_element_type=jnp.float32)
    out_ref[...] = xw2 * dinv


def _fin_body(acc_ref, xwp_ref, dinv_ref, b_ref, wc_ref, bc_ref, out_ref):
    dinv = dinv_ref[...]
    tot = acc_ref[0] + acc_ref[1] + xwp_ref[...]
    h = jnp.maximum(dinv * tot + b_ref[...], 0.0)
    logits = jnp.dot(h, wc_ref[...], preferred_element_type=jnp.float32)
    logits = logits + bc_ref[...]
    m = jnp.max(logits, axis=1, keepdims=True)
    z = logits - m
    lse = jnp.log(jnp.sum(jnp.exp(z), axis=1, keepdims=True))
    out_ref[...] = (z - lse)[:, :C]


def kernel(x, edge_index, E_meta, ptr, w0, b0, conv_W, conv_b, lt1_W, lt1_b):
    del ptr

    packed = (jnp.left_shift(edge_index[0], 16) | edge_index[1])
    edges3 = packed.reshape(NW, NCH, K)
    ones_k = jnp.ones((K,), jnp.float32)
    zeros_k = jnp.zeros((K,), jnp.float32)

    deg_p = _make_deg_kernel()(edges3, ones_k, zeros_k)
    degp4 = deg_p.reshape(NC, G, SEG, 1)

    x3 = x.reshape(G, SEG, F_IN)
    em3 = E_meta.reshape(G, 1, H)
    xwp1, dinv = pl.pallas_call(
        _seg_body,
        grid=(G,),
        in_specs=[
            pl.BlockSpec((1, 1, H), lambda i: (i, 0, 0)),
            pl.BlockSpec((1, SEG, F_IN), lambda i: (i, 0, 0)),
            pl.BlockSpec((NC, 1, SEG, 1), lambda i: (0, i, 0, 0)),
            pl.BlockSpec((F_IN, 1), lambda i: (0, 0)),
            pl.BlockSpec((F_IN, H), lambda i: (0, 0)),
            pl.BlockSpec((H, H), lambda i: (0, 0)),
        ],
        out_specs=[
            pl.BlockSpec((1, SEG, H), lambda i: (i, 0, 0)),
            pl.BlockSpec((1, SEG, 1), lambda i: (i, 0, 0)),
        ],
        out_shape=[
            jax.ShapeDtypeStruct((G, SEG, H), jnp.float32),
            jax.ShapeDtypeStruct((G, SEG, 1), jnp.float32),
        ],
    )(em3, x3, degp4, w0, b0, conv_W[0])
    xwp1 = xwp1.reshape(N, H)
    dinv = dinv.reshape(N, 1)

    RB = 2000
    NRB = N // RB

    acc1 = _make_scatter_kernel()(xwp1, edges3)

    b0row = conv_b[0].reshape(1, H)
    epi_specs = [
        pl.BlockSpec((NC, RB, H), lambda i: (0, i, 0)),
        pl.BlockSpec((RB, H), lambda i: (i, 0)),
        pl.BlockSpec((RB, 1), lambda i: (i, 0)),
        pl.BlockSpec((1, H), lambda i: (0, 0)),
        pl.BlockSpec((H, H), lambda i: (0, 0)),
    ]
    xwp2 = pl.pallas_call(
        _epi_body,
        grid=(NRB,),
        in_specs=epi_specs,
        out_specs=pl.BlockSpec((RB, H), lambda i: (i, 0)),
        out_shape=jax.ShapeDtypeStruct((N, H), jnp.float32),
    )(acc1, xwp1, dinv, b0row, conv_W[1])

    acc2 = _make_scatter_kernel()(xwp2, edges3)

    b1row = conv_b[1].reshape(1, H)
    wc = jnp.pad(lt1_W, ((0, 0), (0, H - C)))
    bc = jnp.pad(lt1_b, (0, H - C), constant_values=-1e30).reshape(1, H)
    out = pl.pallas_call(
        _fin_body,
        grid=(NRB,),
        in_specs=epi_specs + [pl.BlockSpec((1, H), lambda i: (0, 0))],
        out_specs=pl.BlockSpec((RB, C), lambda i: (i, 0)),
        out_shape=jax.ShapeDtypeStruct((N, C), jnp.float32),
    )(acc2, xwp2, dinv, b1row, wc, bc)
    return out
